# Initial kernel scaffold; baseline (speedup 1.0000x reference)
import jax
import jax.numpy as jnp
from jax.experimental import pallas as pl


def kernel(x, edge_index, W1, b1, gamma1, beta1, W2, b2, gamma2, beta2, W3, b3, gamma3, beta3):
    raise NotImplementedError("write your pallas kernel here")


# speedup vs baseline: 8.8067x; 8.8067x over previous
import jax
import jax.numpy as jnp
from jax import lax
from jax.experimental import pallas as pl
from jax.experimental.pallas import tpu as pltpu
from jax.experimental.pallas import tpu_sc as plsc

N = 10000
E = 320000
H = 128
NC = 2
NS = 16
NW = NC * NS
EPW = E // NW
CH = 80
NCH = EPW // CH
NP = 10240
RPT = NP // NS

_MESH = plsc.VectorSubcoreMesh(
    core_axis_name="c", subcore_axis_name="s", num_cores=NC, num_subcores=NS)


def _scat_body(y_hbm, src_hbm, dst_hbm, zeros_hbm, out_hbm,
               src_v, dst_v, rows_v, acc_sh, sem):
    c = lax.axis_index("c")
    s = lax.axis_index("s")
    wid = s * NC + c
    pltpu.sync_copy(zeros_hbm, acc_sh.at[pl.ds(s * RPT, RPT)])
    plsc.subcore_barrier()

    def body(j, carry):
        base = wid * EPW + j * CH
        pltpu.sync_copy(src_hbm.at[pl.ds(base, CH)], src_v)
        pltpu.sync_copy(dst_hbm.at[pl.ds(base, CH)], dst_v)
        pltpu.async_copy(y_hbm.at[src_v], rows_v, sem).wait()
        pltpu.sync_copy(rows_v, acc_sh.at[dst_v], add=True)
        return carry

    lax.fori_loop(0, NCH, body, 0)
    plsc.subcore_barrier()
    pltpu.sync_copy(acc_sh.at[pl.ds(s * RPT, RPT)],
                    out_hbm.at[c, pl.ds(s * RPT, RPT)])


_scat_call = pl.kernel(
    _scat_body,
    out_type=jax.ShapeDtypeStruct((NC, NP, H), jnp.float32),
    mesh=_MESH,
    scratch_types=[
        pltpu.VMEM((CH,), jnp.int32),
        pltpu.VMEM((CH,), jnp.int32),
        pltpu.VMEM((CH, H), jnp.float32),
        pltpu.VMEM_SHARED((NP, H), jnp.float32),
        pltpu.SemaphoreType.DMA,
    ],
)


def _pre_body(x_ref, w_ref, degp_ref, y_ref, dinv_ref):
    deg = 1.0 + degp_ref[0, 0:N, 0:1] + degp_ref[1, 0:N, 0:1]
    dinv = lax.rsqrt(deg)
    dinv_ref[...] = dinv
    y_ref[...] = dinv * jnp.dot(x_ref[...], w_ref[...],
                                preferred_element_type=jnp.float32)


_pre_call = pl.pallas_call(
    _pre_body,
    out_shape=[
        jax.ShapeDtypeStruct((N, H), jnp.float32),
        jax.ShapeDtypeStruct((N, 1), jnp.float32),
    ],
)


def _bn_relu(y_ref, p_ref, dinv_ref, b_ref, g_ref, be_ref):
    dinv = dinv_ref[...]
    z = dinv * (y_ref[...] + p_ref[0, 0:N, :] + p_ref[1, 0:N, :]) + b_ref[...]
    z = jnp.maximum(z, 0.0)
    mu = jnp.mean(z, axis=0, keepdims=True)
    zc = z - mu
    var = jnp.mean(zc * zc, axis=0, keepdims=True)
    return zc * lax.rsqrt(var + 1e-5) * g_ref[...] + be_ref[...]


def _mid_body(y_ref, p_ref, dinv_ref, b_ref, g_ref, be_ref, w_ref, out_ref):
    zn = _bn_relu(y_ref, p_ref, dinv_ref, b_ref, g_ref, be_ref)
    out_ref[...] = dinv_ref[...] * jnp.dot(zn, w_ref[...],
                                           preferred_element_type=jnp.float32)


_mid_call = pl.pallas_call(
    _mid_body,
    out_shape=jax.ShapeDtypeStruct((N, H), jnp.float32),
)


def _post_body(y_ref, p_ref, dinv_ref, b_ref, g_ref, be_ref, out_ref):
    out_ref[...] = _bn_relu(y_ref, p_ref, dinv_ref, b_ref, g_ref, be_ref)


_post_call = pl.pallas_call(
    _post_body,
    out_shape=jax.ShapeDtypeStruct((N, H), jnp.float32),
)


@jax.jit
def kernel(x, edge_index, W1, b1, gamma1, beta1, W2, b2, gamma2, beta2,
           W3, b3, gamma3, beta3):
    src = edge_index[0].astype(jnp.int32)
    dst = edge_index[1].astype(jnp.int32)
    zrows = jnp.zeros((RPT, H), jnp.float32)
    ones_nh = jnp.ones((N, H), jnp.float32)

    degp = _scat_call(ones_nh, src, dst, zrows)
    y1, dinv = _pre_call(x, W1, degp)

    def row(v):
        return v.reshape(1, H)

    p1 = _scat_call(y1, src, dst, zrows)
    y2 = _mid_call(y1, p1, dinv, row(b1), row(gamma1), row(beta1), W2)
    p2 = _scat_call(y2, src, dst, zrows)
    y3 = _mid_call(y2, p2, dinv, row(b2), row(gamma2), row(beta2), W3)
    p3 = _scat_call(y3, src, dst, zrows)
    return _post_call(y3, p3, dinv, row(b3), row(gamma3), row(beta3))


# speedup vs baseline: 19.3291x; 2.1948x over previous
import jax
import jax.numpy as jnp
from jax import lax
from jax.experimental import pallas as pl
from jax.experimental.pallas import tpu as pltpu
from jax.experimental.pallas import tpu_sc as plsc

N = 10000
E = 320000
H = 128
NC = 2
NS = 16
NW = NC * NS
EPW = E // NW
CH = 80
NCH = EPW // CH
NPAIR = NCH // 2
NP = 10240
RPT = NP // NS

_MESH = plsc.VectorSubcoreMesh(
    core_axis_name="c", subcore_axis_name="s", num_cores=NC, num_subcores=NS)


def _scat_body(y_hbm, src_hbm, dst_hbm, zeros_hbm, out_hbm,
               srcA, dstA, srcB, dstB, rowsA, rowsB,
               acc_sh, siA, siB, sgA, sgB):
    c = lax.axis_index("c")
    s = lax.axis_index("s")
    wid = s * NC + c
    e0 = wid * EPW

    def fetch_idx(j, src_v, dst_v, sem):
        pltpu.async_copy(src_hbm.at[pl.ds(e0 + j * CH, CH)], src_v, sem)
        pltpu.async_copy(dst_hbm.at[pl.ds(e0 + j * CH, CH)], dst_v, sem)

    def wait_idx(j, src_v, dst_v, sem):
        pltpu.make_async_copy(src_hbm.at[pl.ds(e0 + j * CH, CH)], src_v,
                              sem).wait()
        pltpu.make_async_copy(dst_hbm.at[pl.ds(e0 + j * CH, CH)], dst_v,
                              sem).wait()

    fetch_idx(0, srcA, dstA, siA)
    pltpu.sync_copy(zeros_hbm, acc_sh.at[pl.ds(s * RPT, RPT)])
    plsc.subcore_barrier()

    def body(jj, carry):
        a = 2 * jj
        wait_idx(a, srcA, dstA, siA)
        ga = pltpu.async_copy(y_hbm.at[srcA], rowsA, sgA)

        @pl.when(jj > 0)
        def _():
            pltpu.sync_copy(rowsB, acc_sh.at[dstB], add=True)

        fetch_idx(a + 1, srcB, dstB, siB)
        wait_idx(a + 1, srcB, dstB, siB)
        gb = pltpu.async_copy(y_hbm.at[srcB], rowsB, sgB)
        ga.wait()
        pltpu.sync_copy(rowsA, acc_sh.at[dstA], add=True)
        fetch_idx(a + 2, srcA, dstA, siA)
        gb.wait()
        return carry

    lax.fori_loop(0, NPAIR, body, 0)
    pltpu.sync_copy(rowsB, acc_sh.at[dstB], add=True)
    wait_idx(NCH - 1, srcA, dstA, siA)
    pltpu.async_copy(y_hbm.at[srcA], rowsA, sgA).wait()
    pltpu.sync_copy(rowsA, acc_sh.at[dstA], add=True)

    plsc.subcore_barrier()
    pltpu.sync_copy(acc_sh.at[pl.ds(s * RPT, RPT)],
                    out_hbm.at[c, pl.ds(s * RPT, RPT)])


_scat_call = pl.kernel(
    _scat_body,
    out_type=jax.ShapeDtypeStruct((NC, NP, H), jnp.float32),
    mesh=_MESH,
    scratch_types=[
        pltpu.VMEM((CH,), jnp.int32),
        pltpu.VMEM((CH,), jnp.int32),
        pltpu.VMEM((CH,), jnp.int32),
        pltpu.VMEM((CH,), jnp.int32),
        pltpu.VMEM((CH, H), jnp.float32),
        pltpu.VMEM((CH, H), jnp.float32),
        pltpu.VMEM_SHARED((NP, H), jnp.float32),
        pltpu.SemaphoreType.DMA,
        pltpu.SemaphoreType.DMA,
        pltpu.SemaphoreType.DMA,
        pltpu.SemaphoreType.DMA,
    ],
)


def _deg_body(dst_hbm, ones_hbm, zeros_hbm, out_hbm,
              dstA, dstB, ones_v, acc_sh, siA, siB):
    c = lax.axis_index("c")
    s = lax.axis_index("s")
    wid = s * NC + c
    e0 = wid * EPW

    def fetch(j, dst_v, sem):
        pltpu.async_copy(dst_hbm.at[pl.ds(e0 + j * CH, CH)], dst_v, sem)

    def wait(j, dst_v, sem):
        pltpu.make_async_copy(dst_hbm.at[pl.ds(e0 + j * CH, CH)], dst_v,
                              sem).wait()

    fetch(0, dstA, siA)
    pltpu.sync_copy(ones_hbm, ones_v)
    pltpu.sync_copy(zeros_hbm, acc_sh.at[pl.ds(s * RPT, RPT)])
    plsc.subcore_barrier()

    def body(jj, carry):
        a = 2 * jj
        wait(a, dstA, siA)
        fetch(a + 1, dstB, siB)
        pltpu.sync_copy(ones_v, acc_sh.at[dstA], add=True)
        wait(a + 1, dstB, siB)
        fetch(a + 2, dstA, siA)
        pltpu.sync_copy(ones_v, acc_sh.at[dstB], add=True)
        return carry

    lax.fori_loop(0, NPAIR, body, 0)
    wait(NCH - 1, dstA, siA)
    pltpu.sync_copy(ones_v, acc_sh.at[dstA], add=True)

    plsc.subcore_barrier()
    pltpu.sync_copy(acc_sh.at[pl.ds(s * RPT, RPT)],
                    out_hbm.at[c, pl.ds(s * RPT, RPT)])


_deg_call = pl.kernel(
    _deg_body,
    out_type=jax.ShapeDtypeStruct((NC, NP, H), jnp.float32),
    mesh=_MESH,
    scratch_types=[
        pltpu.VMEM((CH,), jnp.int32),
        pltpu.VMEM((CH,), jnp.int32),
        pltpu.VMEM((CH, H), jnp.float32),
        pltpu.VMEM_SHARED((NP, H), jnp.float32),
        pltpu.SemaphoreType.DMA,
        pltpu.SemaphoreType.DMA,
    ],
)


def _pre_body(x_ref, w_ref, degp_ref, y_ref, dinv_ref):
    deg = 1.0 + degp_ref[0, 0:N, 0:1] + degp_ref[1, 0:N, 0:1]
    dinv = lax.rsqrt(deg)
    dinv_ref[...] = dinv
    y_ref[...] = dinv * jnp.dot(x_ref[...], w_ref[...],
                                preferred_element_type=jnp.float32)


_pre_call = pl.pallas_call(
    _pre_body,
    out_shape=[
        jax.ShapeDtypeStruct((N, H), jnp.float32),
        jax.ShapeDtypeStruct((N, 1), jnp.float32),
    ],
)


def _bn_relu(y_ref, p_ref, dinv_ref, b_ref, g_ref, be_ref):
    dinv = dinv_ref[...]
    z = dinv * (y_ref[...] + p_ref[0, 0:N, :] + p_ref[1, 0:N, :]) + b_ref[...]
    z = jnp.maximum(z, 0.0)
    mu = jnp.mean(z, axis=0, keepdims=True)
    zc = z - mu
    var = jnp.mean(zc * zc, axis=0, keepdims=True)
    return zc * lax.rsqrt(var + 1e-5) * g_ref[...] + be_ref[...]


def _mid_body(y_ref, p_ref, dinv_ref, b_ref, g_ref, be_ref, w_ref, out_ref):
    zn = _bn_relu(y_ref, p_ref, dinv_ref, b_ref, g_ref, be_ref)
    out_ref[...] = dinv_ref[...] * jnp.dot(zn, w_ref[...],
                                           preferred_element_type=jnp.float32)


_mid_call = pl.pallas_call(
    _mid_body,
    out_shape=jax.ShapeDtypeStruct((N, H), jnp.float32),
)


def _post_body(y_ref, p_ref, dinv_ref, b_ref, g_ref, be_ref, out_ref):
    out_ref[...] = _bn_relu(y_ref, p_ref, dinv_ref, b_ref, g_ref, be_ref)


_post_call = pl.pallas_call(
    _post_body,
    out_shape=jax.ShapeDtypeStruct((N, H), jnp.float32),
)


@jax.jit
def kernel(x, edge_index, W1, b1, gamma1, beta1, W2, b2, gamma2, beta2,
           W3, b3, gamma3, beta3):
    src = edge_index[0].astype(jnp.int32)
    dst = edge_index[1].astype(jnp.int32)
    zrows = jnp.zeros((RPT, H), jnp.float32)
    ones_ch = jnp.ones((CH, H), jnp.float32)

    degp = _deg_call(dst, ones_ch, zrows)
    y1, dinv = _pre_call(x, W1, degp)

    def row(v):
        return v.reshape(1, H)

    p1 = _scat_call(y1, src, dst, zrows)
    y2 = _mid_call(y1, p1, dinv, row(b1), row(gamma1), row(beta1), W2)
    p2 = _scat_call(y2, src, dst, zrows)
    y3 = _mid_call(y2, p2, dinv, row(b2), row(gamma2), row(beta2), W3)
    p3 = _scat_call(y3, src, dst, zrows)
    return _post_call(y3, p3, dinv, row(b3), row(gamma3), row(beta3))


# speedup vs baseline: 21.9123x; 1.1336x over previous
import jax
import jax.numpy as jnp
from jax import lax
from jax.experimental import pallas as pl
from jax.experimental.pallas import tpu as pltpu
from jax.experimental.pallas import tpu_sc as plsc

N = 10000
E = 320000
H = 128
NC = 2
NS = 16
NW = NC * NS
EPW = E // NW
CH = 128
NFC = EPW // CH
TCH = EPW - NFC * CH
NPAIR = NFC // 2
NP = 10240
RPT = NP // NS

_MESH = plsc.VectorSubcoreMesh(
    core_axis_name="c", subcore_axis_name="s", num_cores=NC, num_subcores=NS)


def _scat_body(y_hbm, src_hbm, dst_hbm, zeros_hbm, out_hbm,
               srcA, dstA, srcB, dstB, srcT, dstT, rowsA, rowsB, rowsT,
               acc_sh, siA, siB, sgA, sgB):
    c = lax.axis_index("c")
    s = lax.axis_index("s")
    wid = s * NC + c
    e0 = wid * EPW

    def fetch_idx(j, src_v, dst_v, sem, n=CH):
        pltpu.async_copy(src_hbm.at[pl.ds(e0 + j * CH, n)], src_v, sem)
        pltpu.async_copy(dst_hbm.at[pl.ds(e0 + j * CH, n)], dst_v, sem)

    def wait_idx(j, src_v, dst_v, sem, n=CH):
        pltpu.make_async_copy(src_hbm.at[pl.ds(e0 + j * CH, n)], src_v,
                              sem).wait()
        pltpu.make_async_copy(dst_hbm.at[pl.ds(e0 + j * CH, n)], dst_v,
                              sem).wait()

    fetch_idx(0, srcA, dstA, siA)
    pltpu.sync_copy(zeros_hbm, acc_sh.at[pl.ds(s * RPT, RPT)])
    plsc.subcore_barrier()

    def body(jj, carry):
        a = 2 * jj
        wait_idx(a, srcA, dstA, siA)
        ga = pltpu.async_copy(y_hbm.at[srcA], rowsA, sgA)

        @pl.when(jj > 0)
        def _():
            pltpu.sync_copy(rowsB, acc_sh.at[dstB], add=True)

        fetch_idx(a + 1, srcB, dstB, siB)
        wait_idx(a + 1, srcB, dstB, siB)
        gb = pltpu.async_copy(y_hbm.at[srcB], rowsB, sgB)
        ga.wait()
        pltpu.sync_copy(rowsA, acc_sh.at[dstA], add=True)

        @pl.when(jj + 1 < NPAIR)
        def _():
            fetch_idx(a + 2, srcA, dstA, siA)

        gb.wait()
        return carry

    lax.fori_loop(0, NPAIR, body, 0)
    fetch_idx(NFC, srcT, dstT, siA, n=TCH)
    pltpu.sync_copy(rowsB, acc_sh.at[dstB], add=True)
    wait_idx(NFC, srcT, dstT, siA, n=TCH)
    pltpu.async_copy(y_hbm.at[srcT], rowsT, sgA).wait()
    pltpu.sync_copy(rowsT, acc_sh.at[dstT], add=True)

    plsc.subcore_barrier()
    pltpu.sync_copy(acc_sh.at[pl.ds(s * RPT, RPT)],
                    out_hbm.at[c, pl.ds(s * RPT, RPT)])


_scat_call = pl.kernel(
    _scat_body,
    out_type=jax.ShapeDtypeStruct((NC, NP, H), jnp.float32),
    mesh=_MESH,
    scratch_types=[
        pltpu.VMEM((CH,), jnp.int32),
        pltpu.VMEM((CH,), jnp.int32),
        pltpu.VMEM((CH,), jnp.int32),
        pltpu.VMEM((CH,), jnp.int32),
        pltpu.VMEM((TCH,), jnp.int32),
        pltpu.VMEM((TCH,), jnp.int32),
        pltpu.VMEM((CH, H), jnp.float32),
        pltpu.VMEM((CH, H), jnp.float32),
        pltpu.VMEM((TCH, H), jnp.float32),
        pltpu.VMEM_SHARED((NP, H), jnp.float32),
        pltpu.SemaphoreType.DMA,
        pltpu.SemaphoreType.DMA,
        pltpu.SemaphoreType.DMA,
        pltpu.SemaphoreType.DMA,
    ],
)


def _deg_body(dst_hbm, ones_hbm, zeros_hbm, out_hbm,
              dstA, dstB, dstT, ones_v, acc_sh, siA, siB):
    c = lax.axis_index("c")
    s = lax.axis_index("s")
    wid = s * NC + c
    e0 = wid * EPW

    def fetch(j, dst_v, sem, n=CH):
        pltpu.async_copy(dst_hbm.at[pl.ds(e0 + j * CH, n)], dst_v, sem)

    def wait(j, dst_v, sem, n=CH):
        pltpu.make_async_copy(dst_hbm.at[pl.ds(e0 + j * CH, n)], dst_v,
                              sem).wait()

    fetch(0, dstA, siA)
    pltpu.sync_copy(ones_hbm, ones_v)
    pltpu.sync_copy(zeros_hbm, acc_sh.at[pl.ds(s * RPT, RPT)])
    plsc.subcore_barrier()

    def body(jj, carry):
        a = 2 * jj
        wait(a, dstA, siA)
        fetch(a + 1, dstB, siB)
        pltpu.sync_copy(ones_v, acc_sh.at[dstA], add=True)
        wait(a + 1, dstB, siB)

        @pl.when(jj + 1 < NPAIR)
        def _():
            fetch(a + 2, dstA, siA)

        pltpu.sync_copy(ones_v, acc_sh.at[dstB], add=True)
        return carry

    lax.fori_loop(0, NPAIR, body, 0)
    fetch(NFC, dstT, siA, n=TCH)
    wait(NFC, dstT, siA, n=TCH)
    pltpu.sync_copy(ones_v.at[pl.ds(0, TCH)], acc_sh.at[dstT], add=True)

    plsc.subcore_barrier()
    pltpu.sync_copy(acc_sh.at[pl.ds(s * RPT, RPT)],
                    out_hbm.at[c, pl.ds(s * RPT, RPT)])


_deg_call = pl.kernel(
    _deg_body,
    out_type=jax.ShapeDtypeStruct((NC, NP, H), jnp.float32),
    mesh=_MESH,
    scratch_types=[
        pltpu.VMEM((CH,), jnp.int32),
        pltpu.VMEM((CH,), jnp.int32),
        pltpu.VMEM((TCH,), jnp.int32),
        pltpu.VMEM((CH, H), jnp.float32),
        pltpu.VMEM_SHARED((NP, H), jnp.float32),
        pltpu.SemaphoreType.DMA,
        pltpu.SemaphoreType.DMA,
    ],
)


def _pre_body(x_ref, w_ref, degp_ref, y_ref, dinv_ref):
    deg = 1.0 + degp_ref[0, 0:N, 0:1] + degp_ref[1, 0:N, 0:1]
    dinv = lax.rsqrt(deg)
    dinv_ref[...] = dinv
    y_ref[...] = dinv * jnp.dot(x_ref[...], w_ref[...],
                                preferred_element_type=jnp.float32)


_pre_call = pl.pallas_call(
    _pre_body,
    out_shape=[
        jax.ShapeDtypeStruct((N, H), jnp.float32),
        jax.ShapeDtypeStruct((N, 1), jnp.float32),
    ],
)


def _bn_relu(y_ref, p_ref, dinv_ref, b_ref, g_ref, be_ref):
    dinv = dinv_ref[...]
    z = dinv * (y_ref[...] + p_ref[0, 0:N, :] + p_ref[1, 0:N, :]) + b_ref[...]
    z = jnp.maximum(z, 0.0)
    mu = jnp.mean(z, axis=0, keepdims=True)
    zc = z - mu
    var = jnp.mean(zc * zc, axis=0, keepdims=True)
    return zc * lax.rsqrt(var + 1e-5) * g_ref[...] + be_ref[...]


def _mid_body(y_ref, p_ref, dinv_ref, b_ref, g_ref, be_ref, w_ref, out_ref):
    zn = _bn_relu(y_ref, p_ref, dinv_ref, b_ref, g_ref, be_ref)
    out_ref[...] = dinv_ref[...] * jnp.dot(zn, w_ref[...],
                                           preferred_element_type=jnp.float32)


_mid_call = pl.pallas_call(
    _mid_body,
    out_shape=jax.ShapeDtypeStruct((N, H), jnp.float32),
)


def _post_body(y_ref, p_ref, dinv_ref, b_ref, g_ref, be_ref, out_ref):
    out_ref[...] = _bn_relu(y_ref, p_ref, dinv_ref, b_ref, g_ref, be_ref)


_post_call = pl.pallas_call(
    _post_body,
    out_shape=jax.ShapeDtypeStruct((N, H), jnp.float32),
)


@jax.jit
def kernel(x, edge_index, W1, b1, gamma1, beta1, W2, b2, gamma2, beta2,
           W3, b3, gamma3, beta3):
    src = edge_index[0].astype(jnp.int32)
    dst = edge_index[1].astype(jnp.int32)
    zrows = jnp.zeros((RPT, H), jnp.float32)
    ones_ch = jnp.ones((CH, H), jnp.float32)

    degp = _deg_call(dst, ones_ch, zrows)
    y1, dinv = _pre_call(x, W1, degp)

    def row(v):
        return v.reshape(1, H)

    p1 = _scat_call(y1, src, dst, zrows)
    y2 = _mid_call(y1, p1, dinv, row(b1), row(gamma1), row(beta1), W2)
    p2 = _scat_call(y2, src, dst, zrows)
    y3 = _mid_call(y2, p2, dinv, row(b2), row(gamma2), row(beta2), W3)
    p3 = _scat_call(y3, src, dst, zrows)
    return _post_call(y3, p3, dinv, row(b3), row(gamma3), row(beta3))
